# baseline (speedup 1.0000x reference)
import functools

import jax
import jax.numpy as jnp
from jax import lax
from jax.experimental import pallas as pl
from jax.experimental.pallas import tpu as pltpu
from jax.experimental.pallas import tpu_sc as plsc

_LQ = 4096
_LK = 4096
_BW = 1024
_WSUB = 1280
_BR = 512
_NCH = 2
_CROWS = _LQ // _NCH


def _sc_body(h, table_hbm, delta_hbm, band_hbm, table_v, delta_v, win_v, sem):
    cid = lax.axis_index("c")
    sid = lax.axis_index("s")
    wid = sid * 2 + cid
    residue = wid % 8
    m0 = 256 * h + (wid // 8) * 64

    pltpu.sync_copy(table_hbm, table_v)
    pltpu.sync_copy(delta_hbm, delta_v)
    dvec = delta_v[...]
    dsc = dvec[0]
    iot = lax.broadcasted_iota(jnp.int32, (16,), 0)

    for q in range(2):
        mq = m0 + 32 * q
        i0b = 8 * mq - (8 * mq) % _BR
        cstart = jnp.clip(
            jnp.right_shift(i0b - dsc - 129, 7) * 128, 0, _LK - _BW)
        c0q = iot + (cstart - 120 - residue - 8 * mq) + dvec

        def build(tb, carry, c0q=c0q, q=q):
            idx = jnp.clip(c0q + tb * 16, 0, 256)
            win_v[pl.ds(pl.multiple_of(q * _WSUB + tb * 16, 8), 16)] = \
                plsc.load_gather(table_v, [idx])
            return carry

        lax.fori_loop(0, _WSUB // 16, build, 0)

    loc = residue + 512 * (wid // 8)

    for q in range(2):
        def rows(blk, carry, q=q):
            for j in range(8):
                src_off = pl.multiple_of(
                    q * _WSUB + 248 - 64 * blk - 8 * j, 8)
                dst_off = pl.multiple_of(
                    (loc + 256 * q + 64 * blk + 8 * j) * _BW, 8)
                pltpu.async_copy(
                    win_v.at[pl.ds(src_off, _BW)],
                    band_hbm.at[pl.ds(dst_off, _BW)], sem)
            return carry

        lax.fori_loop(0, 4, rows, 0)

    def drain(blk, carry):
        for _ in range(8):
            pltpu.make_async_copy(
                win_v.at[pl.ds(0, _BW)], band_hbm.at[pl.ds(0, _BW)], sem
            ).wait()
        return carry

    lax.fori_loop(0, 8, drain, 0)


def _make_sc(h):
    mesh = plsc.VectorSubcoreMesh(core_axis_name="c", subcore_axis_name="s")
    return pl.kernel(
        functools.partial(_sc_body, h),
        out_type=jax.ShapeDtypeStruct((_CROWS * _BW,), jnp.float32),
        mesh=mesh,
        compiler_params=pltpu.CompilerParams(needs_layout_passes=False),
        scratch_types=[
            pltpu.VMEM((272,), jnp.float32),
            pltpu.VMEM((16,), jnp.int32),
            pltpu.VMEM((2 * _WSUB,), jnp.float32),
            pltpu.SemaphoreType.DMA,
        ],
    )


_SC_CALLS = [_make_sc(h) for h in range(_NCH)]

_STEPS = _CROWS // _BR


def _tc_block(i0, dsm_ref, tsm_ref, band_ref, out_ref):
    delta = dsm_ref[0]
    c_lo = tsm_ref[0]
    c_hi = tsm_ref[256]
    cstart = jnp.clip(
        jnp.right_shift(i0 - delta - 129, 7) * 128, 0, _LK - _BW)

    cols_f = lax.broadcasted_iota(jnp.int32, (_BR, _LK), 1)
    out_ref[...] = jnp.where(cols_f < cstart + _BW // 2, c_lo, c_hi)

    band = band_ref[...].reshape(_BR, _BW)
    rows_w = i0 + lax.broadcasted_iota(jnp.int32, (_BR, _BW), 0)
    cols_w = cstart + lax.broadcasted_iota(jnp.int32, (_BR, _BW), 1)
    d_w = cols_w - rows_w + delta
    mixed = jnp.where(d_w <= -128, c_lo, jnp.where(d_w >= 128, c_hi, band))
    out_ref[:, pl.ds(pl.multiple_of(cstart, 128), _BW)] = mixed


def _tc_first_body(dsm_ref, tsm_ref, band_ref, out_ref):
    _tc_block(pl.program_id(0) * _BR, dsm_ref, tsm_ref, band_ref, out_ref)


def _tc_chunk_body(h, dsm_ref, tsm_ref, band_ref, carry_ref, out_ref):
    del carry_ref
    _tc_block((pl.program_id(0) + h * _STEPS) * _BR,
              dsm_ref, tsm_ref, band_ref, out_ref)


def _tc_first(delta_arr, table_p, band3):
    return pl.pallas_call(
        _tc_first_body,
        grid=(_STEPS,),
        in_specs=[
            pl.BlockSpec(memory_space=pltpu.SMEM),
            pl.BlockSpec(memory_space=pltpu.SMEM),
            pl.BlockSpec((_BR, _BW // 128, 128), lambda i: (i, 0, 0)),
        ],
        out_specs=pl.BlockSpec((_BR, _LK), lambda i: (i, 0)),
        out_shape=jax.ShapeDtypeStruct((_LQ, _LK), jnp.float32),
    )(delta_arr, table_p, band3)


def _make_tc(h):
    def f(delta_arr, table_p, band3, carry):
        return pl.pallas_call(
            functools.partial(_tc_chunk_body, h),
            grid=(_STEPS,),
            in_specs=[
                pl.BlockSpec(memory_space=pltpu.SMEM),
                pl.BlockSpec(memory_space=pltpu.SMEM),
                pl.BlockSpec((_BR, _BW // 128, 128), lambda i: (i, 0, 0)),
                pl.BlockSpec(memory_space=pl.ANY),
            ],
            out_specs=pl.BlockSpec(
                (_BR, _LK), lambda i, h=h: (i + h * _STEPS, 0)),
            out_shape=jax.ShapeDtypeStruct((_LQ, _LK), jnp.float32),
            input_output_aliases={3: 0},
        )(delta_arr, table_p, band3, carry)
    return f


_TC_CALLS = [_tc_first] + [_make_tc(h) for h in range(1, _NCH)]


@jax.jit
def _rel_pos(table_p, delta_arr):
    bands = [sc(table_p, delta_arr) for sc in _SC_CALLS]
    out = _TC_CALLS[0](
        delta_arr, table_p, bands[0].reshape(_CROWS, _BW // 128, 128))
    for h in range(1, _NCH):
        out = _TC_CALLS[h](
            delta_arr, table_p,
            bands[h].reshape(_CROWS, _BW // 128, 128), out)
    return out


def kernel(embeddings_table, length_q, length_k):
    delta = (length_k - _LK) - (length_q - _LQ)
    table_p = jnp.pad(embeddings_table.astype(jnp.float32), (0, 15))
    delta_arr = jnp.full((16,), delta, dtype=jnp.int32)
    return _rel_pos(table_p, delta_arr)


# speedup vs baseline: 1.0990x; 1.0990x over previous
import functools

import jax
import jax.numpy as jnp
from jax import lax
from jax.experimental import pallas as pl
from jax.experimental.pallas import tpu as pltpu
from jax.experimental.pallas import tpu_sc as plsc

_LQ = 4096
_LK = 4096
_BW = 1024
_WSUB = 1280
_BR = 512


def _sc_body(table_hbm, delta_hbm, band_hbm, table_v, delta_v, win_v, sem):
    cid = lax.axis_index("c")
    sid = lax.axis_index("s")
    wid = sid * 2 + cid
    residue = wid % 8
    m0 = (wid // 8) * 128

    pltpu.sync_copy(table_hbm, table_v)
    pltpu.sync_copy(delta_hbm, delta_v)
    dvec = delta_v[...]
    dsc = dvec[0]
    iot = lax.broadcasted_iota(jnp.int32, (16,), 0)

    for q in range(4):
        mq = m0 + 32 * q
        i0b = 8 * mq - (8 * mq) % _BR
        cstart = jnp.clip(
            jnp.right_shift(i0b - dsc - 129, 7) * 128, 0, _LK - _BW)
        c0q = iot + (cstart - 120 - residue - 8 * mq) + dvec

        def build(tb, carry, c0q=c0q, q=q):
            idx = jnp.clip(c0q + tb * 16, 0, 256)
            win_v[pl.ds(pl.multiple_of(q * _WSUB + tb * 16, 8), 16)] = \
                plsc.load_gather(table_v, [idx])
            return carry

        lax.fori_loop(0, _WSUB // 16, build, 0)

    for q in range(4):
        def rows(blk, carry, q=q):
            for j in range(8):
                src_off = pl.multiple_of(
                    q * _WSUB + 248 - 64 * blk - 8 * j, 8)
                dst_off = pl.multiple_of(
                    (residue + 8 * (m0 + 32 * q + 8 * blk + j)) * _BW, 8)
                pltpu.async_copy(
                    win_v.at[pl.ds(src_off, _BW)],
                    band_hbm.at[pl.ds(dst_off, _BW)], sem)
            return carry

        lax.fori_loop(0, 4, rows, 0)

    def drain(blk, carry):
        for _ in range(8):
            pltpu.make_async_copy(
                win_v.at[pl.ds(0, _BW)], band_hbm.at[pl.ds(0, _BW)], sem
            ).wait()
        return carry

    lax.fori_loop(0, 16, drain, 0)


def _sc_call(table_p, delta_arr):
    mesh = plsc.VectorSubcoreMesh(core_axis_name="c", subcore_axis_name="s")
    return pl.kernel(
        _sc_body,
        out_type=jax.ShapeDtypeStruct((_LQ * _BW,), jnp.float32),
        mesh=mesh,
        compiler_params=pltpu.CompilerParams(needs_layout_passes=False),
        scratch_types=[
            pltpu.VMEM((272,), jnp.float32),
            pltpu.VMEM((16,), jnp.int32),
            pltpu.VMEM((4 * _WSUB,), jnp.float32),
            pltpu.SemaphoreType.DMA,
        ],
    )(table_p, delta_arr)


def _tc_body(dsm_ref, tsm_ref, band_ref, out_ref):
    i0 = pl.program_id(0) * _BR
    delta = dsm_ref[0]
    c_lo = tsm_ref[0]
    c_hi = tsm_ref[256]
    cstart = jnp.clip(
        jnp.right_shift(i0 - delta - 129, 7) * 128, 0, _LK - _BW)

    cols_f = lax.broadcasted_iota(jnp.int32, (_BR, _LK), 1)
    out_ref[...] = jnp.where(cols_f < cstart + _BW // 2, c_lo, c_hi)

    band = band_ref[...].reshape(_BR, _BW)
    rows_w = i0 + lax.broadcasted_iota(jnp.int32, (_BR, _BW), 0)
    cols_w = cstart + lax.broadcasted_iota(jnp.int32, (_BR, _BW), 1)
    d_w = cols_w - rows_w + delta
    mixed = jnp.where(d_w <= -128, c_lo, jnp.where(d_w >= 128, c_hi, band))
    out_ref[:, pl.ds(pl.multiple_of(cstart, 128), _BW)] = mixed


def _tc_call(delta_arr, table_p, band3):
    return pl.pallas_call(
        _tc_body,
        grid=(_LQ // _BR,),
        in_specs=[
            pl.BlockSpec(memory_space=pltpu.SMEM),
            pl.BlockSpec(memory_space=pltpu.SMEM),
            pl.BlockSpec((_BR, _BW // 128, 128), lambda i: (i, 0, 0)),
        ],
        out_specs=pl.BlockSpec((_BR, _LK), lambda i: (i, 0)),
        out_shape=jax.ShapeDtypeStruct((_LQ, _LK), jnp.float32),
    )(delta_arr, table_p, band3)


@jax.jit
def _rel_pos(table_p, delta_arr):
    band = _sc_call(table_p, delta_arr)
    return _tc_call(delta_arr, table_p, band.reshape(_LQ, _BW // 128, 128))


def kernel(embeddings_table, length_q, length_k):
    delta = (length_k - _LK) - (length_q - _LQ)
    table_p = jnp.pad(embeddings_table.astype(jnp.float32), (0, 15))
    delta_arr = jnp.full((16,), delta, dtype=jnp.int32)
    return _rel_pos(table_p, delta_arr)


# speedup vs baseline: 1.1074x; 1.0077x over previous
import functools

import jax
import jax.numpy as jnp
from jax import lax
from jax.experimental import pallas as pl
from jax.experimental.pallas import tpu as pltpu
from jax.experimental.pallas import tpu_sc as plsc

_LQ = 4096
_LK = 4096
_BW = 1024
_WSUB = 1280
_BR = 512


def _sc_body(table_hbm, delta_hbm, band_hbm, table_v, delta_v, win_v, sem):
    cid = lax.axis_index("c")
    sid = lax.axis_index("s")
    wid = sid * 2 + cid
    residue = wid % 8
    m0 = (wid // 8) * 128

    pltpu.sync_copy(table_hbm, table_v)
    pltpu.sync_copy(delta_hbm, delta_v)
    dvec = delta_v[...]
    dsc = dvec[0]
    iot = lax.broadcasted_iota(jnp.int32, (16,), 0)

    for q in range(4):
        mq = m0 + 32 * q
        i0b = 8 * mq - (8 * mq) % _BR
        cstart = jnp.clip(
            jnp.right_shift(i0b - dsc - 129, 7) * 128, 0, _LK - _BW)
        c0q = iot + (cstart - 120 - residue - 8 * mq) + dvec

        def build(tb, carry, c0q=c0q, q=q):
            idx = jnp.clip(c0q + tb * 16, 0, 256)
            win_v[pl.ds(pl.multiple_of(q * _WSUB + tb * 16, 8), 16)] = \
                plsc.load_gather(table_v, [idx])
            return carry

        lax.fori_loop(0, _WSUB // 16, build, 0)

    for q in range(4):
        def rows(blk, carry, q=q):
            for j in range(8):
                src_off = pl.multiple_of(
                    q * _WSUB + 248 - 64 * blk - 8 * j, 8)
                dst_off = pl.multiple_of(
                    (residue + 8 * (m0 + 32 * q + 8 * blk + j)) * _BW, 8)
                pltpu.async_copy(
                    win_v.at[pl.ds(src_off, _BW)],
                    band_hbm.at[pl.ds(dst_off, _BW)], sem)
            return carry

        lax.fori_loop(0, 4, rows, 0)

    def drain(blk, carry):
        for _ in range(8):
            pltpu.make_async_copy(
                win_v.at[pl.ds(0, _BW)], band_hbm.at[pl.ds(0, _BW)], sem
            ).wait()
        return carry

    lax.fori_loop(0, 16, drain, 0)


def _sc_call(table_p, delta_arr):
    mesh = plsc.VectorSubcoreMesh(core_axis_name="c", subcore_axis_name="s")
    return pl.kernel(
        _sc_body,
        out_type=jax.ShapeDtypeStruct((_LQ * _BW,), jnp.float32),
        mesh=mesh,
        compiler_params=pltpu.CompilerParams(needs_layout_passes=False),
        scratch_types=[
            pltpu.VMEM((257,), jnp.float32),
            pltpu.VMEM((16,), jnp.int32),
            pltpu.VMEM((4 * _WSUB,), jnp.float32),
            pltpu.SemaphoreType.DMA,
        ],
    )(table_p, delta_arr)


def _tc_body(dsm_ref, tsm_ref, band_ref, out_ref):
    i0 = pl.program_id(0) * _BR
    delta = dsm_ref[0]
    c_lo = tsm_ref[0]
    c_hi = tsm_ref[256]
    cstart = jnp.clip(
        jnp.right_shift(i0 - delta - 129, 7) * 128, 0, _LK - _BW)

    cols_f = lax.broadcasted_iota(jnp.int32, (_BR, _LK), 1)
    out_ref[...] = jnp.where(cols_f < cstart + _BW // 2, c_lo, c_hi)

    band = band_ref[...].reshape(_BR, _BW)
    rows_w = i0 + lax.broadcasted_iota(jnp.int32, (_BR, _BW), 0)
    cols_w = cstart + lax.broadcasted_iota(jnp.int32, (_BR, _BW), 1)
    d_w = cols_w - rows_w + delta
    mixed = jnp.where(d_w <= -128, c_lo, jnp.where(d_w >= 128, c_hi, band))
    out_ref[:, pl.ds(pl.multiple_of(cstart, 128), _BW)] = mixed


def _tc_call(delta_arr, table_p, band3):
    return pl.pallas_call(
        _tc_body,
        grid=(_LQ // _BR,),
        in_specs=[
            pl.BlockSpec(memory_space=pltpu.SMEM),
            pl.BlockSpec(memory_space=pltpu.SMEM),
            pl.BlockSpec((_BR, _BW // 128, 128), lambda i: (i, 0, 0)),
        ],
        out_specs=pl.BlockSpec((_BR, _LK), lambda i: (i, 0)),
        out_shape=jax.ShapeDtypeStruct((_LQ, _LK), jnp.float32),
    )(delta_arr, table_p, band3)


@jax.jit
def _rel_pos(table_p, delta_arr):
    band = _sc_call(table_p, delta_arr)
    return _tc_call(delta_arr, table_p, band.reshape(_LQ, _BW // 128, 128))


def kernel(embeddings_table, length_q, length_k):
    delta = (length_k - _LK) - (length_q - _LQ)
    table_p = embeddings_table.astype(jnp.float32)
    delta_arr = jnp.full((16,), delta, dtype=jnp.int32)
    return _rel_pos(table_p, delta_arr)
